# Initial kernel scaffold; baseline (speedup 1.0000x reference)
import jax
import jax.numpy as jnp
from jax.experimental import pallas as pl


def kernel(input_ids, s, emb_table, W1, b1, W2, b2):
    raise NotImplementedError("write your pallas kernel here")


# speedup vs baseline: 62.5658x; 62.5658x over previous
import functools

import jax
import jax.numpy as jnp
from jax import lax
from jax.experimental import pallas as pl
from jax.experimental.pallas import tpu as pltpu
from jax.experimental.pallas import tpu_sc as plsc

_NC = 2
_NS = 16
_NW = _NC * _NS


def _tc_vocab_scalar(table, W1, b1_2d, w2a, b2_2d):
    vocab, emb = table.shape
    blk = 8000
    assert vocab % blk == 0

    def body(x_ref, w1_ref, b1_ref, w2_ref, b2_ref, o_ref):
        x = x_ref[...]
        z = jnp.dot(x, w1_ref[...], preferred_element_type=jnp.float32)
        z = jnp.maximum(z + b1_ref[...], 0.0)
        o_ref[...] = (
            jnp.dot(z, w2_ref[...], preferred_element_type=jnp.float32)
            + b2_ref[...]
        )

    return pl.pallas_call(
        body,
        grid=(vocab // blk,),
        in_specs=[
            pl.BlockSpec((blk, emb), lambda i: (i, 0)),
            pl.BlockSpec((emb, 8), lambda i: (0, 0)),
            pl.BlockSpec((1, 8), lambda i: (0, 0)),
            pl.BlockSpec((8, 1), lambda i: (0, 0)),
            pl.BlockSpec((1, 1), lambda i: (0, 0)),
        ],
        out_specs=pl.BlockSpec((blk, 1), lambda i: (i, 0)),
        out_shape=jax.ShapeDtypeStruct((vocab, 1), jnp.float32),
    )(table, W1, b1_2d, w2a, b2_2d)


def _sc_gather_axpy(v1d, ids2d, s2d, cvec):
    rows = ids2d.shape[0]
    rows_per_w = rows // _NW
    ch_rows = 16
    nchunk = rows_per_w // ch_rows
    assert rows_per_w % ch_rows == 0

    mesh = plsc.VectorSubcoreMesh(core_axis_name="c", subcore_axis_name="s")

    @functools.partial(
        pl.kernel,
        out_type=jax.ShapeDtypeStruct((rows, 128), jnp.float32),
        mesh=mesh,
        scratch_types=[
            pltpu.VMEM((ch_rows, 128), jnp.int32),
            pltpu.VMEM((ch_rows, 128), jnp.float32),
            pltpu.VMEM((ch_rows, 128), jnp.float32),
            pltpu.VMEM((16,), jnp.float32),
            pltpu.SemaphoreType.DMA,
        ],
    )
    def sc_k(v_hbm, ids_hbm, s_hbm, c_hbm, out_hbm, idx_b, val_b, s_b, c_b, sem):
        wid = lax.axis_index("s") * _NC + lax.axis_index("c")
        pltpu.sync_copy(c_hbm, c_b)
        cv = c_b[...]
        base = wid * rows_per_w

        def chunk_body(ci, carry):
            r0 = base + ci * ch_rows
            pltpu.sync_copy(ids_hbm.at[pl.ds(r0, ch_rows)], idx_b)
            copies = [
                pltpu.async_copy(v_hbm.at[idx_b.at[j]], val_b.at[j], sem)
                for j in range(ch_rows)
            ]
            pltpu.sync_copy(s_hbm.at[pl.ds(r0, ch_rows)], s_b)
            for c in copies:
                c.wait()

            def row_body(j, c2):
                for i in range(8):
                    sl = pl.ds(i * 16, 16)
                    val_b[j, sl] = val_b[j, sl] + s_b[j, sl] * cv
                return c2

            lax.fori_loop(0, ch_rows, row_body, 0)
            pltpu.sync_copy(val_b, out_hbm.at[pl.ds(r0, ch_rows)])
            return carry

        lax.fori_loop(0, nchunk, chunk_body, 0)

    return sc_k(v1d, ids2d, s2d, cvec)


def kernel(input_ids, s, emb_table, W1, b1, W2, b2):
    B, _, L = input_ids.shape
    vocab = emb_table.shape[0]
    bl = B * L
    assert bl % (128 * _NW) == 0

    v = _tc_vocab_scalar(
        emb_table, W1, b1.reshape(1, 8), W2[:8], b2.reshape(1, 1)
    )
    v1d = v.reshape(vocab)
    ids2d = input_ids.reshape(bl // 128, 128)
    s2d = s.reshape(bl // 128, 128)
    cvec = jnp.full((16,), W2[8, 0], dtype=jnp.float32)

    out2d = _sc_gather_axpy(v1d, ids2d, s2d, cvec)
    return out2d.reshape(B, L)


# speedup vs baseline: 87.6006x; 1.4001x over previous
import functools

import jax
import jax.numpy as jnp
from jax import lax
from jax.experimental import pallas as pl
from jax.experimental.pallas import tpu as pltpu
from jax.experimental.pallas import tpu_sc as plsc

_NC = 2
_NS = 16
_NW = _NC * _NS


def _tc_vocab_scalar(table, W1, b1_2d, w2a, b2_2d, vocab_pad):
    vocab, emb = table.shape
    blk = 8192
    grid = pl.cdiv(vocab, blk)

    def body(x_ref, w1_ref, b1_ref, w2_ref, b2_ref, o_ref):
        x = x_ref[...]
        z = jnp.dot(x, w1_ref[...], preferred_element_type=jnp.float32)
        z = jnp.maximum(z + b1_ref[...], 0.0)
        vcol = (
            jnp.dot(z, w2_ref[...], preferred_element_type=jnp.float32)
            + b2_ref[...]
        )
        o_ref[...] = vcol.reshape(blk // 128, 128)

    return pl.pallas_call(
        body,
        grid=(grid,),
        in_specs=[
            pl.BlockSpec((blk, emb), lambda i: (i, 0)),
            pl.BlockSpec((emb, 8), lambda i: (0, 0)),
            pl.BlockSpec((1, 8), lambda i: (0, 0)),
            pl.BlockSpec((8, 1), lambda i: (0, 0)),
            pl.BlockSpec((1, 1), lambda i: (0, 0)),
        ],
        out_specs=pl.BlockSpec((blk // 128, 128), lambda i: (i, 0)),
        out_shape=jax.ShapeDtypeStruct((vocab_pad // 128, 128), jnp.float32),
    )(table, W1, b1_2d, w2a, b2_2d)


def _sc_gather_axpy(v1d, ids2d, s1d, cvec):
    rows = ids2d.shape[0]
    n = s1d.shape[0]
    rows_per_w = rows // _NW
    ch_rows = 16
    nchunk = rows_per_w // ch_rows
    assert rows_per_w % ch_rows == 0 and n == rows * 128

    mesh = plsc.VectorSubcoreMesh(core_axis_name="c", subcore_axis_name="s")

    @functools.partial(
        pl.kernel,
        out_type=jax.ShapeDtypeStruct((n,), jnp.float32),
        mesh=mesh,
        scratch_types=[
            pltpu.VMEM((ch_rows, 128), jnp.int32),
            pltpu.VMEM((ch_rows * 128,), jnp.float32),
            pltpu.VMEM((ch_rows * 128,), jnp.float32),
            pltpu.VMEM((16,), jnp.float32),
            pltpu.SemaphoreType.DMA,
        ],
    )
    def sc_k(v_hbm, ids_hbm, s_hbm, c_hbm, out_hbm, idx_b, val_b, s_b, c_b, sem):
        wid = lax.axis_index("s") * _NC + lax.axis_index("c")
        pltpu.sync_copy(c_hbm, c_b)
        cv = c_b[...]
        base = wid * rows_per_w

        def chunk_body(ci, carry):
            r0 = base + ci * ch_rows
            f0 = r0 * 128
            pltpu.sync_copy(ids_hbm.at[pl.ds(r0, ch_rows)], idx_b)
            copies = [
                pltpu.async_copy(
                    v_hbm.at[idx_b.at[j]], val_b.at[pl.ds(j * 128, 128)], sem
                )
                for j in range(ch_rows)
            ]
            pltpu.sync_copy(s_hbm.at[pl.ds(f0, ch_rows * 128)], s_b)
            for c in copies:
                c.wait()

            def vec_body(k, c2):
                sl = pl.ds(k * 16, 16)
                val_b[sl] = val_b[sl] + s_b[sl] * cv
                return c2

            lax.fori_loop(0, ch_rows * 8, vec_body, 0)
            pltpu.sync_copy(val_b, out_hbm.at[pl.ds(f0, ch_rows * 128)])
            return carry

        lax.fori_loop(0, nchunk, chunk_body, 0)

    return sc_k(v1d, ids2d, s1d, cvec)


def kernel(input_ids, s, emb_table, W1, b1, W2, b2):
    B, _, L = input_ids.shape
    vocab = emb_table.shape[0]
    bl = B * L
    assert bl % (128 * _NW) == 0
    vocab_pad = ((vocab + 8191) // 8192) * 8192

    v2d = _tc_vocab_scalar(
        emb_table, W1, b1.reshape(1, 8), W2[:8], b2.reshape(1, 1), vocab_pad
    )
    v1d = v2d.reshape(vocab_pad)
    ids2d = input_ids.reshape(bl // 128, 128)
    s1d = s.reshape(bl)
    cvec = jnp.full((16,), W2[8, 0], dtype=jnp.float32)

    out1d = _sc_gather_axpy(v1d, ids2d, s1d, cvec)
    return out1d.reshape(B, L)


# speedup vs baseline: 103.1737x; 1.1778x over previous
import functools

import jax
import jax.numpy as jnp
from jax import lax
from jax.experimental import pallas as pl
from jax.experimental.pallas import tpu as pltpu
from jax.experimental.pallas import tpu_sc as plsc

_NC = 2
_NS = 16
_NW = _NC * _NS


def _tc_vocab_scalar(table, W1, b1_2d, w2a, b2_2d, vocab_pad):
    vocab, emb = table.shape
    blk = 8192
    grid = pl.cdiv(vocab, blk)

    def body(x_ref, w1_ref, b1_ref, w2_ref, b2_ref, o_ref):
        x = x_ref[...]
        z = jnp.dot(x, w1_ref[...], preferred_element_type=jnp.float32)
        z = jnp.maximum(z + b1_ref[...], 0.0)
        vcol = (
            jnp.dot(z, w2_ref[...], preferred_element_type=jnp.float32)
            + b2_ref[...]
        )
        o_ref[...] = vcol.reshape(blk // 128, 128)

    return pl.pallas_call(
        body,
        grid=(grid,),
        in_specs=[
            pl.BlockSpec((blk, emb), lambda i: (i, 0)),
            pl.BlockSpec((emb, 8), lambda i: (0, 0)),
            pl.BlockSpec((1, 8), lambda i: (0, 0)),
            pl.BlockSpec((8, 1), lambda i: (0, 0)),
            pl.BlockSpec((1, 1), lambda i: (0, 0)),
        ],
        out_specs=pl.BlockSpec((blk // 128, 128), lambda i: (i, 0)),
        out_shape=jax.ShapeDtypeStruct((vocab_pad // 128, 128), jnp.float32),
    )(table, W1, b1_2d, w2a, b2_2d)


def _sc_gather_axpy(v1d, ids2d, s1d, cvec):
    rows = ids2d.shape[0]
    n = s1d.shape[0]
    rows_per_w = rows // _NW
    nb = 8
    nchunk = rows_per_w // nb
    depth = 4
    pf = 2
    ntok = nb * 128
    assert rows_per_w % nb == 0 and nchunk % depth == 0 and n == rows * 128

    mesh = plsc.VectorSubcoreMesh(core_axis_name="c", subcore_axis_name="s")
    ring = lambda ty: [ty] * depth

    @functools.partial(
        pl.kernel,
        out_type=jax.ShapeDtypeStruct((n,), jnp.float32),
        mesh=mesh,
        scratch_types=[
            *ring(pltpu.VMEM((nb, 128), jnp.int32)),
            *ring(pltpu.VMEM((ntok,), jnp.float32)),
            *ring(pltpu.VMEM((ntok,), jnp.float32)),
            *ring(pltpu.VMEM((ntok,), jnp.float32)),
            pltpu.VMEM((16,), jnp.float32),
            *ring(pltpu.SemaphoreType.DMA),
            *ring(pltpu.SemaphoreType.DMA),
            *ring(pltpu.SemaphoreType.DMA),
            *ring(pltpu.SemaphoreType.DMA),
        ],
    )
    def sc_k(v_hbm, ids_hbm, s_hbm, c_hbm, out_hbm, *bufs):
        idx_b = bufs[0:depth]
        val_b = bufs[depth : 2 * depth]
        s_b = bufs[2 * depth : 3 * depth]
        out_b = bufs[3 * depth : 4 * depth]
        c_b = bufs[4 * depth]
        isem = bufs[4 * depth + 1 : 5 * depth + 1]
        gsem = bufs[5 * depth + 1 : 6 * depth + 1]
        ssem = bufs[6 * depth + 1 : 7 * depth + 1]
        osem = bufs[7 * depth + 1 : 8 * depth + 1]

        wid = lax.axis_index("s") * _NC + lax.axis_index("c")
        r_base = wid * rows_per_w
        f_base = r_base * 128
        pltpu.sync_copy(c_hbm, c_b)
        cv = c_b[...]

        def drain(ref, sem):
            pltpu.make_async_copy(v_hbm.at[pl.ds(0, ntok)], ref, sem).wait()

        def drain_idx(u):
            pltpu.make_async_copy(
                ids_hbm.at[pl.ds(0, nb)], idx_b[u], isem[u]
            ).wait()

        def fire_idx(c, u):
            pltpu.async_copy(
                ids_hbm.at[pl.ds(r_base + c * nb, nb)], idx_b[u], isem[u]
            )

        def fire(c, u):
            drain_idx(u)
            for j in range(nb):
                pltpu.async_copy(
                    v_hbm.at[idx_b[u].at[j]],
                    val_b[u].at[pl.ds(j * 128, 128)],
                    gsem[u],
                )
            pltpu.async_copy(
                s_hbm.at[pl.ds(f_base + c * ntok, ntok)], s_b[u], ssem[u]
            )

        def process(c, u, head=False):
            drain(val_b[u], gsem[u])
            drain(s_b[u], ssem[u])
            for k in range(ntok // 16):
                sl = pl.ds(k * 16, 16)
                out_b[u][sl] = val_b[u][sl] + s_b[u][sl] * cv
            pltpu.async_copy(
                out_b[u], out_hbm.at[pl.ds(f_base + c * ntok, ntok)], osem[u]
            )
            fire_idx(jnp.minimum(c + depth, nchunk - 1), u)
            cp = c + pf
            up = (u + pf) % depth
            if not head:
                drain(out_b[up], osem[up])
            fire(jnp.minimum(cp, nchunk - 1), up)

        for u in range(depth):
            fire_idx(u, u)
        for u in range(pf):
            fire(u, u)
        for u in range(depth):
            process(u, u, head=u < pf)

        def group(g, carry):
            for u in range(depth):
                process(g * depth + u, u)
            return carry

        lax.fori_loop(1, nchunk // depth, group, 0)
        for u in (0, 1):
            drain(val_b[u], gsem[u])
            drain(s_b[u], ssem[u])
        for u in (2, 3):
            drain_idx(u)
            drain(out_b[u], osem[u])

    return sc_k(v1d, ids2d, s1d, cvec)


def kernel(input_ids, s, emb_table, W1, b1, W2, b2):
    B, _, L = input_ids.shape
    vocab = emb_table.shape[0]
    bl = B * L
    assert bl % (128 * _NW) == 0
    vocab_pad = ((vocab + 8191) // 8192) * 8192

    v2d = _tc_vocab_scalar(
        emb_table, W1, b1.reshape(1, 8), W2[:8], b2.reshape(1, 1), vocab_pad
    )
    v1d = v2d.reshape(vocab_pad)
    ids2d = input_ids.reshape(bl // 128, 128)
    s1d = s.reshape(bl)
    cvec = jnp.full((16,), W2[8, 0], dtype=jnp.float32)

    out1d = _sc_gather_axpy(v1d, ids2d, s1d, cvec)
    return out1d.reshape(B, L)


# speedup vs baseline: 111.4474x; 1.0802x over previous
import functools

import jax
import jax.numpy as jnp
from jax import lax
from jax.experimental import pallas as pl
from jax.experimental.pallas import tpu as pltpu
from jax.experimental.pallas import tpu_sc as plsc

_NC = 2
_NS = 16
_NW = _NC * _NS


def _tc_vocab_scalar(table, W1, b1_2d, w2a, b2_2d, vocab_pad):
    vocab, emb = table.shape
    blk = 16384
    grid = pl.cdiv(vocab, blk)

    def body(x_ref, w1_ref, b1_ref, w2_ref, b2_ref, o_ref):
        x = x_ref[...]
        z = jnp.dot(x, w1_ref[...], preferred_element_type=jnp.float32)
        z = jnp.maximum(z + b1_ref[...], 0.0)
        vcol = (
            jnp.dot(z, w2_ref[...], preferred_element_type=jnp.float32)
            + b2_ref[...]
        )
        o_ref[...] = vcol.reshape(blk // 128, 128)

    return pl.pallas_call(
        body,
        grid=(grid,),
        in_specs=[
            pl.BlockSpec((blk, emb), lambda i: (i, 0)),
            pl.BlockSpec((emb, 8), lambda i: (0, 0)),
            pl.BlockSpec((1, 8), lambda i: (0, 0)),
            pl.BlockSpec((8, 1), lambda i: (0, 0)),
            pl.BlockSpec((1, 1), lambda i: (0, 0)),
        ],
        out_specs=pl.BlockSpec((blk // 128, 128), lambda i: (i, 0)),
        out_shape=jax.ShapeDtypeStruct((vocab_pad // 128, 128), jnp.float32),
    )(table, W1, b1_2d, w2a, b2_2d)


def _sc_gather_axpy(v1d, ids2d, s1d, cvec):
    rows = ids2d.shape[0]
    n = s1d.shape[0]
    rows_per_w = rows // _NW
    nb = 8
    nchunk = rows_per_w // nb
    depth = 4
    pf = 2
    ntok = nb * 128
    assert rows_per_w % nb == 0 and nchunk % depth == 0 and n == rows * 128

    mesh = plsc.VectorSubcoreMesh(core_axis_name="c", subcore_axis_name="s")
    ring = lambda ty: [ty] * depth

    @functools.partial(
        pl.kernel,
        out_type=jax.ShapeDtypeStruct((n,), jnp.float32),
        mesh=mesh,
        scratch_types=[
            *ring(pltpu.VMEM((nb, 128), jnp.int32)),
            *ring(pltpu.VMEM((ntok,), jnp.float32)),
            *ring(pltpu.VMEM((ntok,), jnp.float32)),
            *ring(pltpu.VMEM((ntok,), jnp.float32)),
            pltpu.VMEM((16,), jnp.float32),
            *ring(pltpu.SemaphoreType.DMA),
            *ring(pltpu.SemaphoreType.DMA),
            *ring(pltpu.SemaphoreType.DMA),
            *ring(pltpu.SemaphoreType.DMA),
        ],
    )
    def sc_k(v_hbm, ids_hbm, s_hbm, c_hbm, out_hbm, *bufs):
        idx_b = bufs[0:depth]
        val_b = bufs[depth : 2 * depth]
        s_b = bufs[2 * depth : 3 * depth]
        out_b = bufs[3 * depth : 4 * depth]
        c_b = bufs[4 * depth]
        isem = bufs[4 * depth + 1 : 5 * depth + 1]
        gsem = bufs[5 * depth + 1 : 6 * depth + 1]
        ssem = bufs[6 * depth + 1 : 7 * depth + 1]
        osem = bufs[7 * depth + 1 : 8 * depth + 1]

        wid = lax.axis_index("s") * _NC + lax.axis_index("c")
        r_base = wid * rows_per_w
        f_base = r_base * 128
        pltpu.sync_copy(c_hbm, c_b)
        cv = c_b[...]

        def drain(ref, sem, src):
            pltpu.make_async_copy(src, ref, sem).wait()

        def drain_idx(u):
            drain(idx_b[u], isem[u], ids_hbm.at[pl.ds(0, nb)])

        def fire_idx(c, u):
            pltpu.async_copy(
                ids_hbm.at[pl.ds(r_base + c * nb, nb)], idx_b[u], isem[u]
            )

        def fire(c, u):
            drain_idx(u)
            for j in range(nb):
                pltpu.async_copy(
                    v_hbm.at[idx_b[u].at[j]],
                    val_b[u].at[pl.ds(j * 128, 128)],
                    gsem[u],
                )
            pltpu.async_copy(
                s_hbm.at[pl.ds(f_base + c * ntok, ntok)], s_b[u], ssem[u]
            )

        def process(c, u, head=False):
            drain(val_b[u], gsem[u], v_hbm.at[pl.ds(0, ntok)])
            drain(s_b[u], ssem[u], s_hbm.at[pl.ds(0, ntok)])
            for k in range(ntok // 16):
                sl = pl.ds(k * 16, 16)
                out_b[u][sl] = val_b[u][sl] + s_b[u][sl] * cv
            pltpu.async_copy(
                out_b[u], out_hbm.at[pl.ds(f_base + c * ntok, ntok)], osem[u]
            )
            fire_idx(jnp.minimum(c + depth, nchunk - 1), u)
            cp = c + pf
            up = (u + pf) % depth
            if not head:
                drain(out_b[up], osem[up], s_hbm.at[pl.ds(0, ntok)])
            fire(jnp.minimum(cp, nchunk - 1), up)

        for u in range(depth):
            fire_idx(u, u)
        for u in range(pf):
            fire(u, u)
        for u in range(depth):
            process(u, u, head=u < pf)

        def group(g, carry):
            for u in range(depth):
                process(g * depth + u, u)
            return carry

        lax.fori_loop(1, nchunk // depth, group, 0)
        for u in (0, 1):
            drain(val_b[u], gsem[u], v_hbm.at[pl.ds(0, ntok)])
            drain(s_b[u], ssem[u], s_hbm.at[pl.ds(0, ntok)])
        for u in (2, 3):
            drain_idx(u)
            drain(out_b[u], osem[u], s_hbm.at[pl.ds(0, ntok)])

    return sc_k(v1d, ids2d, s1d, cvec)


def kernel(input_ids, s, emb_table, W1, b1, W2, b2):
    B, _, L = input_ids.shape
    vocab = emb_table.shape[0]
    bl = B * L
    assert bl % (128 * _NW) == 0
    vocab_pad = ((vocab + 16383) // 16384) * 16384

    v2d = _tc_vocab_scalar(
        emb_table, W1, b1.reshape(1, 8), W2[:8], b2.reshape(1, 1), vocab_pad
    )
    v1d = v2d.reshape(vocab_pad)
    ids2d = input_ids.reshape(bl // 128, 128)
    s1d = s.reshape(bl)
    cvec = jnp.full((16,), W2[8, 0], dtype=jnp.float32)

    out1d = _sc_gather_axpy(v1d, ids2d, s1d, cvec)
    return out1d.reshape(B, L)
